# Initial kernel scaffold; baseline (speedup 1.0000x reference)
import jax
import jax.numpy as jnp
from jax.experimental import pallas as pl


def kernel(edge_attr, W0, W1, W2, W3):
    raise NotImplementedError("write your pallas kernel here")


# speedup vs baseline: 2.0680x; 2.0680x over previous
import functools

import jax
import jax.numpy as jnp
from jax import lax
from jax.experimental import pallas as pl
from jax.experimental.pallas import tpu as pltpu
from jax.experimental.pallas import tpu_sc as plsc

EMB = 128
E = 320000
ROWS = E // EMB
NLUT = 128

NC = 2
NS = 16
NW = NC * NS
EPW = E // NW
CHUNK = 80
NCH = EPW // CHUNK


def _prep_body(ea_ref, w0_ref, w1_ref, w2_ref, w3_ref, combo_ref, lut_ref):
    k = lax.broadcasted_iota(jnp.int32, (4 * EMB, EMB), 0)
    g = lax.broadcasted_iota(jnp.int32, (4 * EMB, EMB), 1)
    km = k % 4
    stride = jnp.where(km == 0, 24, jnp.where(km == 1, 4, jnp.where(km == 2, 2, 1)))
    s = jnp.where(k // 4 == g, stride, 0).astype(jnp.float32)
    ea = ea_ref[...].astype(jnp.float32)
    combo = jax.lax.dot(ea, s, preferred_element_type=jnp.float32)
    combo_ref[...] = combo.astype(jnp.int32)

    c = lax.broadcasted_iota(jnp.int32, (NLUT, 1), 0)
    i0 = c // 24
    i1 = (c // 4) % 6
    i2 = (c // 2) % 2
    i3 = c % 2
    lut = jnp.zeros((NLUT, EMB), jnp.float32)
    for j in range(5):
        lut = lut + jnp.where(i0 == j, 1.0, 0.0) * w0_ref[j, :][None, :]
    for j in range(6):
        lut = lut + jnp.where(i1 == j, 1.0, 0.0) * w1_ref[j, :][None, :]
    for j in range(2):
        lut = lut + jnp.where(i2 == j, 1.0, 0.0) * w2_ref[j, :][None, :]
        lut = lut + jnp.where(i3 == j, 1.0, 0.0) * w3_ref[j, :][None, :]
    lut_ref[...] = lut


_prep = pl.pallas_call(
    _prep_body,
    out_shape=(
        jax.ShapeDtypeStruct((ROWS, EMB), jnp.int32),
        jax.ShapeDtypeStruct((NLUT, EMB), jnp.float32),
    ),
)

@functools.cache
def _make_sc_gather():
    mesh = plsc.VectorSubcoreMesh(core_axis_name="c", subcore_axis_name="s")

    @functools.partial(
        pl.kernel,
        mesh=mesh,
        out_type=jax.ShapeDtypeStruct((E, EMB), jnp.float32),
        scratch_types=[
            pltpu.VMEM((CHUNK,), jnp.int32),
            pltpu.VMEM((CHUNK, EMB), jnp.float32),
            pltpu.SemaphoreType.DMA,
        ],
    )
    def _sc_gather(lut_hbm, combo_hbm, out_hbm, idx_v, rows_v, sem):
        wid = lax.axis_index("s") * NC + lax.axis_index("c")
        base0 = wid * EPW

        def step(j, carry):
            base = base0 + j * CHUNK
            pltpu.sync_copy(combo_hbm.at[pl.ds(base, CHUNK)], idx_v)
            pltpu.async_copy(lut_hbm.at[idx_v], rows_v, sem).wait()
            pltpu.sync_copy(rows_v, out_hbm.at[pl.ds(base, CHUNK)])
            return carry

        lax.fori_loop(0, NCH, step, 0)

    return _sc_gather


def kernel(edge_attr, W0, W1, W2, W3):
    ea = edge_attr.astype(jnp.int32).reshape(ROWS, 4 * EMB)
    combo2d, lut = _prep(ea, W0, W1, W2, W3)
    return _make_sc_gather()(lut, combo2d.reshape(E))


# speedup vs baseline: 2.0900x; 1.0107x over previous
import functools

import jax
import jax.numpy as jnp
from jax import lax
from jax.experimental import pallas as pl
from jax.experimental.pallas import tpu as pltpu
from jax.experimental.pallas import tpu_sc as plsc

EMB = 128
E = 320000
ROWS = E // EMB
NLUT = 128

NC = 2
NS = 16
NW = NC * NS
EPW = E // NW
CHUNK = 80
NCH = EPW // CHUNK


def _prep_body(ea_ref, w0_ref, w1_ref, w2_ref, w3_ref, combo_ref, lut_ref):
    k = lax.broadcasted_iota(jnp.int32, (4 * EMB, EMB), 0)
    g = lax.broadcasted_iota(jnp.int32, (4 * EMB, EMB), 1)
    km = k % 4
    stride = jnp.where(km == 0, 24, jnp.where(km == 1, 4, jnp.where(km == 2, 2, 1)))
    s = jnp.where(k // 4 == g, stride, 0).astype(jnp.float32)
    ea = ea_ref[...].astype(jnp.float32)
    combo = jax.lax.dot(ea, s, preferred_element_type=jnp.float32)
    combo_ref[...] = combo.astype(jnp.int32)

    c = lax.broadcasted_iota(jnp.int32, (NLUT, 1), 0)
    i0 = c // 24
    i1 = (c // 4) % 6
    i2 = (c // 2) % 2
    i3 = c % 2
    lut = jnp.zeros((NLUT, EMB), jnp.float32)
    for j in range(5):
        lut = lut + jnp.where(i0 == j, 1.0, 0.0) * w0_ref[j, :][None, :]
    for j in range(6):
        lut = lut + jnp.where(i1 == j, 1.0, 0.0) * w1_ref[j, :][None, :]
    for j in range(2):
        lut = lut + jnp.where(i2 == j, 1.0, 0.0) * w2_ref[j, :][None, :]
        lut = lut + jnp.where(i3 == j, 1.0, 0.0) * w3_ref[j, :][None, :]
    lut_ref[...] = lut


_prep = pl.pallas_call(
    _prep_body,
    out_shape=(
        jax.ShapeDtypeStruct((ROWS, EMB), jnp.int32),
        jax.ShapeDtypeStruct((NLUT, EMB), jnp.float32),
    ),
)

@functools.cache
def _make_sc_gather():
    mesh = plsc.VectorSubcoreMesh(core_axis_name="c", subcore_axis_name="s")

    @functools.partial(
        pl.kernel,
        mesh=mesh,
        out_type=jax.ShapeDtypeStruct((E, EMB), jnp.float32),
        scratch_types=[
            pltpu.VMEM((2, CHUNK), jnp.int32),
            pltpu.VMEM((2, CHUNK, EMB), jnp.float32),
            pltpu.SemaphoreType.DMA((2,)),
            pltpu.SemaphoreType.DMA((2,)),
        ],
    )
    def _sc_gather(lut_hbm, combo_hbm, out_hbm, idx_v, rows_v, gsem, ssem):
        wid = lax.axis_index("s") * NC + lax.axis_index("c")
        base0 = wid * EPW

        def fetch_and_gather(j, b):
            base = base0 + j * CHUNK
            pltpu.sync_copy(combo_hbm.at[pl.ds(base, CHUNK)], idx_v.at[b])
            pltpu.async_copy(lut_hbm.at[idx_v.at[b]], rows_v.at[b], gsem.at[b])

        fetch_and_gather(0, 0)

        def step(j, carry):
            b = j % 2
            nb = (j + 1) % 2

            @pl.when(j + 1 < NCH)
            def _():
                @pl.when(j >= 1)
                def _():
                    pltpu.make_async_copy(
                        rows_v.at[nb],
                        out_hbm.at[pl.ds(base0 + (j - 1) * CHUNK, CHUNK)],
                        ssem.at[nb],
                    ).wait()

                fetch_and_gather(j + 1, nb)

            pltpu.make_async_copy(
                lut_hbm.at[idx_v.at[b]], rows_v.at[b], gsem.at[b]
            ).wait()
            pltpu.async_copy(
                rows_v.at[b], out_hbm.at[pl.ds(base0 + j * CHUNK, CHUNK)], ssem.at[b]
            )
            return carry

        lax.fori_loop(0, NCH, step, 0)

        for j in (NCH - 2, NCH - 1):
            b = j % 2
            pltpu.make_async_copy(
                rows_v.at[b],
                out_hbm.at[pl.ds(base0 + j * CHUNK, CHUNK)],
                ssem.at[b],
            ).wait()

    return _sc_gather


def kernel(edge_attr, W0, W1, W2, W3):
    ea = edge_attr.astype(jnp.int32).reshape(ROWS, 4 * EMB)
    combo2d, lut = _prep(ea, W0, W1, W2, W3)
    return _make_sc_gather()(lut, combo2d.reshape(E))


# speedup vs baseline: 2.1164x; 1.0126x over previous
import functools

import jax
import jax.numpy as jnp
from jax import lax
from jax.experimental import pallas as pl
from jax.experimental.pallas import tpu as pltpu
from jax.experimental.pallas import tpu_sc as plsc

EMB = 128
E = 320000
ROWS = E // EMB
NLUT = 128

NC = 2
NS = 16
NW = NC * NS
EPW = E // NW
CHUNK = 400
NCH = EPW // CHUNK


def _prep_body(ea_ref, w0_ref, w1_ref, w2_ref, w3_ref, combo_ref, lut_ref):
    k = lax.broadcasted_iota(jnp.int32, (4 * EMB, EMB), 0)
    g = lax.broadcasted_iota(jnp.int32, (4 * EMB, EMB), 1)
    km = k % 4
    stride = jnp.where(km == 0, 24, jnp.where(km == 1, 4, jnp.where(km == 2, 2, 1)))
    s = jnp.where(k // 4 == g, stride, 0).astype(jnp.float32)
    ea = ea_ref[...].astype(jnp.float32)
    combo = jax.lax.dot(ea, s, preferred_element_type=jnp.float32)
    combo_ref[...] = combo.astype(jnp.int32)

    c = lax.broadcasted_iota(jnp.int32, (NLUT, 1), 0)
    i0 = c // 24
    i1 = (c // 4) % 6
    i2 = (c // 2) % 2
    i3 = c % 2
    lut = jnp.zeros((NLUT, EMB), jnp.float32)
    for j in range(5):
        lut = lut + jnp.where(i0 == j, 1.0, 0.0) * w0_ref[j, :][None, :]
    for j in range(6):
        lut = lut + jnp.where(i1 == j, 1.0, 0.0) * w1_ref[j, :][None, :]
    for j in range(2):
        lut = lut + jnp.where(i2 == j, 1.0, 0.0) * w2_ref[j, :][None, :]
        lut = lut + jnp.where(i3 == j, 1.0, 0.0) * w3_ref[j, :][None, :]
    lut_ref[...] = lut


_prep = pl.pallas_call(
    _prep_body,
    out_shape=(
        jax.ShapeDtypeStruct((ROWS, EMB), jnp.int32),
        jax.ShapeDtypeStruct((NLUT, EMB), jnp.float32),
    ),
)

@functools.cache
def _make_sc_gather():
    mesh = plsc.VectorSubcoreMesh(core_axis_name="c", subcore_axis_name="s")

    @functools.partial(
        pl.kernel,
        mesh=mesh,
        out_type=jax.ShapeDtypeStruct((E, EMB), jnp.float32),
        scratch_types=[
            pltpu.VMEM((EPW,), jnp.int32),
            pltpu.VMEM((2, CHUNK, EMB), jnp.float32),
            pltpu.SemaphoreType.DMA((2,)),
            pltpu.SemaphoreType.DMA((2,)),
        ],
    )
    def _sc_gather(lut_hbm, combo_hbm, out_hbm, idx_v, rows_v, gsem, ssem):
        wid = lax.axis_index("s") * NC + lax.axis_index("c")
        base0 = wid * EPW

        pltpu.sync_copy(combo_hbm.at[pl.ds(base0, EPW)], idx_v)

        def fetch_and_gather(j, b):
            pltpu.async_copy(
                lut_hbm.at[idx_v.at[pl.ds(j * CHUNK, CHUNK)]], rows_v.at[b], gsem.at[b]
            )

        fetch_and_gather(0, 0)

        def step(j, carry):
            b = j % 2
            nb = (j + 1) % 2

            @pl.when(j + 1 < NCH)
            def _():
                @pl.when(j >= 1)
                def _():
                    pltpu.make_async_copy(
                        rows_v.at[nb],
                        out_hbm.at[pl.ds(base0 + (j - 1) * CHUNK, CHUNK)],
                        ssem.at[nb],
                    ).wait()

                fetch_and_gather(j + 1, nb)

            pltpu.make_async_copy(
                lut_hbm.at[idx_v.at[pl.ds(j * CHUNK, CHUNK)]], rows_v.at[b], gsem.at[b]
            ).wait()
            pltpu.async_copy(
                rows_v.at[b], out_hbm.at[pl.ds(base0 + j * CHUNK, CHUNK)], ssem.at[b]
            )
            return carry

        lax.fori_loop(0, NCH, step, 0)

        for j in (NCH - 2, NCH - 1):
            b = j % 2
            pltpu.make_async_copy(
                rows_v.at[b],
                out_hbm.at[pl.ds(base0 + j * CHUNK, CHUNK)],
                ssem.at[b],
            ).wait()

    return _sc_gather


def kernel(edge_attr, W0, W1, W2, W3):
    ea = edge_attr.astype(jnp.int32).reshape(ROWS, 4 * EMB)
    combo2d, lut = _prep(ea, W0, W1, W2, W3)
    return _make_sc_gather()(lut, combo2d.reshape(E))


# speedup vs baseline: 5.0436x; 2.3831x over previous
import functools

import jax
import jax.numpy as jnp
from jax import lax
from jax.experimental import pallas as pl
from jax.experimental.pallas import tpu as pltpu
from jax.experimental.pallas import tpu_sc as plsc

EMB = 128
E = 320000
ROWS = E // EMB
NLUT = 128

NC = 2
NS = 16
NW = NC * NS
EPW = E // NW
CHUNK = 400
NCH = EPW // CHUNK


def _prep_body(ea_ref, w0_ref, w1_ref, w2_ref, w3_ref, combo_ref, lut_ref):
    k = lax.broadcasted_iota(jnp.int32, (4 * EMB, EMB), 0)
    g = lax.broadcasted_iota(jnp.int32, (4 * EMB, EMB), 1)
    km = k % 4
    stride = jnp.where(km == 0, 24, jnp.where(km == 1, 4, jnp.where(km == 2, 2, 1)))
    s = jnp.where(k // 4 == g, stride, 0).astype(jnp.float32)
    ea = ea_ref[...].astype(jnp.float32)
    combo = jax.lax.dot(ea, s, preferred_element_type=jnp.float32)
    combo_ref[...] = combo.astype(jnp.int32)

    c = lax.broadcasted_iota(jnp.int32, (NLUT, 1), 0)
    i0 = c // 24
    i1 = (c // 4) % 6
    i2 = (c // 2) % 2
    i3 = c % 2
    lut = jnp.zeros((NLUT, EMB), jnp.float32)
    for j in range(5):
        lut = lut + jnp.where(i0 == j, 1.0, 0.0) * w0_ref[j, :][None, :]
    for j in range(6):
        lut = lut + jnp.where(i1 == j, 1.0, 0.0) * w1_ref[j, :][None, :]
    for j in range(2):
        lut = lut + jnp.where(i2 == j, 1.0, 0.0) * w2_ref[j, :][None, :]
        lut = lut + jnp.where(i3 == j, 1.0, 0.0) * w3_ref[j, :][None, :]
    lut_ref[...] = lut


_prep = pl.pallas_call(
    _prep_body,
    out_shape=(
        jax.ShapeDtypeStruct((ROWS, EMB), jnp.int32),
        jax.ShapeDtypeStruct((NLUT, EMB), jnp.float32),
    ),
)

@functools.cache
def _make_sc_gather():
    mesh = plsc.VectorSubcoreMesh(core_axis_name="c", subcore_axis_name="s")

    @functools.partial(
        pl.kernel,
        mesh=mesh,
        out_type=jax.ShapeDtypeStruct((E, EMB), jnp.float32),
        scratch_types=[
            pltpu.VMEM((NLUT, EMB), jnp.float32),
            pltpu.VMEM((EPW,), jnp.int32),
            pltpu.VMEM((CHUNK, EMB), jnp.float32),
            pltpu.VMEM((CHUNK, EMB), jnp.float32),
            pltpu.SemaphoreType.DMA((2,)),
        ],
    )
    def _sc_gather(lut_hbm, combo_hbm, out_hbm, lut_v, idx_v, rows0, rows1, ssem):
        wid = lax.axis_index("s") * NC + lax.axis_index("c")
        base0 = wid * EPW

        pltpu.sync_copy(lut_hbm, lut_v)
        pltpu.sync_copy(combo_hbm.at[pl.ds(base0, EPW)], idx_v)

        def compute(j, rows_ref):
            def body(q, carry):
                cvec = idx_v[pl.ds(j * CHUNK + q * 16, 16)]
                for l in range(16):
                    c = cvec[l]
                    for k in range(8):
                        rows_ref[q * 16 + l, pl.ds(k * 16, 16)] = lut_v[
                            c, pl.ds(k * 16, 16)
                        ]
                return carry

            lax.fori_loop(0, CHUNK // 16, body, 0)

        def scat(j, rows_ref, b):
            return pltpu.make_async_copy(
                rows_ref, out_hbm.at[pl.ds(base0 + j * CHUNK, CHUNK)], ssem.at[b]
            )

        def pair(p, carry):
            for b, rows_ref in ((0, rows0), (1, rows1)):
                j = 2 * p + b

                @pl.when(p >= 1)
                def _():
                    scat(j - 2, rows_ref, b).wait()

                compute(j, rows_ref)
                scat(j, rows_ref, b).start()
            return carry

        lax.fori_loop(0, NCH // 2, pair, 0)

        jt = NCH - 1
        scat(jt - 2, rows0, 0).wait()
        compute(jt, rows0)
        scat(jt, rows0, 0).start()
        scat(NCH - 2, rows1, 1).wait()
        scat(jt, rows0, 0).wait()

    return _sc_gather


def kernel(edge_attr, W0, W1, W2, W3):
    ea = edge_attr.astype(jnp.int32).reshape(ROWS, 4 * EMB)
    combo2d, lut = _prep(ea, W0, W1, W2, W3)
    return _make_sc_gather()(lut, combo2d.reshape(E))


# speedup vs baseline: 8.3111x; 1.6479x over previous
import functools

import jax
import jax.numpy as jnp
from jax import lax
from jax.experimental import pallas as pl
from jax.experimental.pallas import tpu as pltpu
from jax.experimental.pallas import tpu_sc as plsc

EMB = 128
E = 320000
ROWS = E // EMB
NLUT = 128

NC = 2
NS = 16
NW = NC * NS
EPW = E // NW
CHUNK = 400
NCH = EPW // CHUNK


def _prep_body(ea_ref, w0_ref, w1_ref, w2_ref, w3_ref, combo_ref, lut_ref):
    k = lax.broadcasted_iota(jnp.int32, (4 * EMB, EMB), 0)
    g = lax.broadcasted_iota(jnp.int32, (4 * EMB, EMB), 1)
    km = k % 4
    stride = jnp.where(km == 0, 24, jnp.where(km == 1, 4, jnp.where(km == 2, 2, 1)))
    s = jnp.where(k // 4 == g, stride, 0).astype(jnp.float32)
    ea = ea_ref[...].astype(jnp.float32)
    combo = jax.lax.dot(ea, s, preferred_element_type=jnp.float32)
    combo_ref[...] = combo.astype(jnp.int32)

    c = lax.broadcasted_iota(jnp.int32, (NLUT, 1), 0)
    i0 = c // 24
    i1 = (c // 4) % 6
    i2 = (c // 2) % 2
    i3 = c % 2
    lut = jnp.zeros((NLUT, EMB), jnp.float32)
    for j in range(5):
        lut = lut + jnp.where(i0 == j, 1.0, 0.0) * w0_ref[j, :][None, :]
    for j in range(6):
        lut = lut + jnp.where(i1 == j, 1.0, 0.0) * w1_ref[j, :][None, :]
    for j in range(2):
        lut = lut + jnp.where(i2 == j, 1.0, 0.0) * w2_ref[j, :][None, :]
        lut = lut + jnp.where(i3 == j, 1.0, 0.0) * w3_ref[j, :][None, :]
    lut_ref[...] = lut


_prep = pl.pallas_call(
    _prep_body,
    out_shape=(
        jax.ShapeDtypeStruct((ROWS, EMB), jnp.int32),
        jax.ShapeDtypeStruct((NLUT, EMB), jnp.float32),
    ),
)

@functools.cache
def _make_sc_gather():
    mesh = plsc.VectorSubcoreMesh(core_axis_name="c", subcore_axis_name="s")

    @functools.partial(
        pl.kernel,
        mesh=mesh,
        out_type=jax.ShapeDtypeStruct((E, EMB), jnp.float32),
        scratch_types=[
            pltpu.VMEM((NLUT * EMB,), jnp.float32),
            pltpu.VMEM((EPW,), jnp.int32),
            pltpu.VMEM((CHUNK, EMB), jnp.float32),
            pltpu.VMEM((CHUNK, EMB), jnp.float32),
            pltpu.SemaphoreType.DMA((2,)),
        ],
    )
    def _sc_gather(lut_hbm, combo_hbm, out_hbm, lut_v, idx_v, rows0, rows1, ssem):
        wid = lax.axis_index("s") * NC + lax.axis_index("c")
        base0 = wid * EPW

        pltpu.sync_copy(lut_hbm, lut_v)
        pltpu.sync_copy(combo_hbm.at[pl.ds(base0, EPW)], idx_v)

        def compute(j, rows_ref):
            @plsc.parallel_loop(0, CHUNK // 16)
            def body(q):
                cvec = idx_v[pl.ds(j * CHUNK + q * 16, 16)] * EMB
                for l in range(16):
                    cb = cvec[l]
                    for k in range(8):
                        rows_ref[q * 16 + l, pl.ds(k * 16, 16)] = lut_v[
                            pl.ds(cb + k * 16, 16)
                        ]

        def scat(j, rows_ref, b):
            return pltpu.make_async_copy(
                rows_ref, out_hbm.at[pl.ds(base0 + j * CHUNK, CHUNK)], ssem.at[b]
            )

        def pair(p, carry):
            for b, rows_ref in ((0, rows0), (1, rows1)):
                j = 2 * p + b

                @pl.when(p >= 1)
                def _():
                    scat(j - 2, rows_ref, b).wait()

                compute(j, rows_ref)
                scat(j, rows_ref, b).start()
            return carry

        lax.fori_loop(0, NCH // 2, pair, 0)

        jt = NCH - 1
        scat(jt - 2, rows0, 0).wait()
        compute(jt, rows0)
        scat(jt, rows0, 0).start()
        scat(NCH - 2, rows1, 1).wait()
        scat(jt, rows0, 0).wait()

    return _sc_gather


def kernel(edge_attr, W0, W1, W2, W3):
    ea = edge_attr.astype(jnp.int32).reshape(ROWS, 4 * EMB)
    combo2d, lut = _prep(ea, W0, W1, W2, W3)
    return _make_sc_gather()(lut.reshape(NLUT * EMB), combo2d.reshape(E))


# speedup vs baseline: 9.1842x; 1.1051x over previous
import functools

import jax
import jax.numpy as jnp
from jax import lax
from jax.experimental import pallas as pl
from jax.experimental.pallas import tpu as pltpu
from jax.experimental.pallas import tpu_sc as plsc

EMB = 128
E = 320000
ROWS = E // EMB
NLUT = 128

NC = 2
NS = 16
NW = NC * NS
EPW = E // NW
CHUNK = 400
NCH = EPW // CHUNK


def _prep_body(w0_ref, w1_ref, w2_ref, w3_ref, lut_ref):
    c = lax.broadcasted_iota(jnp.int32, (NLUT, 1), 0)
    i0 = c // 24
    i1 = (c // 4) % 6
    i2 = (c // 2) % 2
    i3 = c % 2
    lut = jnp.zeros((NLUT, EMB), jnp.float32)
    for j in range(5):
        lut = lut + jnp.where(i0 == j, 1.0, 0.0) * w0_ref[j, :][None, :]
    for j in range(6):
        lut = lut + jnp.where(i1 == j, 1.0, 0.0) * w1_ref[j, :][None, :]
    for j in range(2):
        lut = lut + jnp.where(i2 == j, 1.0, 0.0) * w2_ref[j, :][None, :]
        lut = lut + jnp.where(i3 == j, 1.0, 0.0) * w3_ref[j, :][None, :]
    lut_ref[...] = lut


_prep = pl.pallas_call(
    _prep_body,
    out_shape=jax.ShapeDtypeStruct((NLUT, EMB), jnp.float32),
)

@functools.cache
def _make_sc_gather():
    mesh = plsc.VectorSubcoreMesh(core_axis_name="c", subcore_axis_name="s")

    @functools.partial(
        pl.kernel,
        mesh=mesh,
        out_type=jax.ShapeDtypeStruct((E, EMB), jnp.float32),
        scratch_types=[
            pltpu.VMEM((NLUT * EMB,), jnp.float32),
            pltpu.VMEM((CHUNK * 4,), jnp.int32),
            pltpu.VMEM((CHUNK * 4,), jnp.int32),
            pltpu.VMEM((CHUNK, EMB), jnp.float32),
            pltpu.VMEM((CHUNK, EMB), jnp.float32),
            pltpu.SemaphoreType.DMA((2,)),
            pltpu.SemaphoreType.DMA((2,)),
        ],
    )
    def _sc_gather(lut_hbm, ea_hbm, out_hbm, lut_v, ea0, ea1, rows0, rows1, isem, ssem):
        ea_bufs = (ea0, ea1)
        wid = lax.axis_index("s") * NC + lax.axis_index("c")
        base0 = wid * EPW

        pltpu.sync_copy(lut_hbm, lut_v)

        lane = lax.iota(jnp.int32, 16) % 4
        svec = jnp.where(
            lane == 0,
            24 * EMB,
            jnp.where(lane == 1, 4 * EMB, jnp.where(lane == 2, 2 * EMB, EMB)),
        )

        def fetch(j, b):
            return pltpu.make_async_copy(
                ea_hbm.at[pl.ds((base0 + j * CHUNK) * 4, CHUNK * 4)],
                ea_bufs[b],
                isem.at[b],
            )

        def compute(j, b, rows_ref):
            @plsc.parallel_loop(0, CHUNK // 4)
            def body(m):
                p = ea_bufs[b][pl.ds(m * 16, 16)] * svec
                for t in range(4):
                    cb = p[4 * t] + p[4 * t + 1] + p[4 * t + 2] + p[4 * t + 3]
                    for k in range(8):
                        rows_ref[m * 4 + t, pl.ds(k * 16, 16)] = lut_v[
                            pl.ds(cb + k * 16, 16)
                        ]

        def scat(j, rows_ref, b):
            return pltpu.make_async_copy(
                rows_ref, out_hbm.at[pl.ds(base0 + j * CHUNK, CHUNK)], ssem.at[b]
            )

        fetch(0, 0).start()

        def step(j, b, rows_ref, last):
            fetch(j, b).wait()
            if not last:
                fetch(j + 1, 1 - b).start()

            @pl.when(j >= 2)
            def _():
                scat(j - 2, rows_ref, b).wait()

            compute(j, b, rows_ref)
            scat(j, rows_ref, b).start()

        def pair(p, carry):
            j0 = 2 * p
            step(j0, 0, rows0, False)
            step(j0 + 1, 1, rows1, False)
            return carry

        lax.fori_loop(0, NCH // 2, pair, 0)

        step(NCH - 1, 0, rows0, True)
        scat(NCH - 2, rows1, 1).wait()
        scat(NCH - 1, rows0, 0).wait()

    return _sc_gather


def kernel(edge_attr, W0, W1, W2, W3):
    lut = _prep(W0, W1, W2, W3)
    ea_flat = edge_attr.astype(jnp.int32).reshape(E * 4)
    return _make_sc_gather()(lut.reshape(NLUT * EMB), ea_flat)


# speedup vs baseline: 20.6494x; 2.2484x over previous
import functools

import jax
import jax.numpy as jnp
from jax import lax
from jax.experimental import pallas as pl
from jax.experimental.pallas import tpu as pltpu
from jax.experimental.pallas import tpu_sc as plsc

EMB = 128
E = 320000
ROWS = E // EMB
NLUT = 128

NC = 2
NS = 16
NW = NC * NS
EPW = E // NW
CHUNK = 400
NCH = EPW // CHUNK


def _prep_body(w0_ref, w1_ref, w2_ref, w3_ref, lut_ref):
    c = lax.broadcasted_iota(jnp.int32, (NLUT, 1), 0)
    i0 = c // 24
    i1 = (c // 4) % 6
    i2 = (c // 2) % 2
    i3 = c % 2
    lut = jnp.zeros((NLUT, EMB), jnp.float32)
    for j in range(5):
        lut = lut + jnp.where(i0 == j, 1.0, 0.0) * w0_ref[j, :][None, :]
    for j in range(6):
        lut = lut + jnp.where(i1 == j, 1.0, 0.0) * w1_ref[j, :][None, :]
    for j in range(2):
        lut = lut + jnp.where(i2 == j, 1.0, 0.0) * w2_ref[j, :][None, :]
        lut = lut + jnp.where(i3 == j, 1.0, 0.0) * w3_ref[j, :][None, :]
    lut_ref[...] = lut


_prep = pl.pallas_call(
    _prep_body,
    out_shape=jax.ShapeDtypeStruct((NLUT, EMB), jnp.float32),
)

@functools.cache
def _make_sc_gather():
    mesh = plsc.VectorSubcoreMesh(core_axis_name="c", subcore_axis_name="s")

    @functools.partial(
        pl.kernel,
        mesh=mesh,
        out_type=jax.ShapeDtypeStruct((E, EMB), jnp.float32),
        scratch_types=[
            pltpu.VMEM((NLUT * EMB,), jnp.float32),
            pltpu.VMEM((CHUNK,), jnp.int32),
            pltpu.VMEM((CHUNK,), jnp.int32),
            pltpu.VMEM((CHUNK,), jnp.int32),
            pltpu.VMEM((CHUNK,), jnp.int32),
            pltpu.VMEM((CHUNK,), jnp.int32),
            pltpu.VMEM((CHUNK,), jnp.int32),
            pltpu.VMEM((CHUNK,), jnp.int32),
            pltpu.VMEM((CHUNK,), jnp.int32),
            pltpu.VMEM((CHUNK, EMB), jnp.float32),
            pltpu.VMEM((CHUNK, EMB), jnp.float32),
            pltpu.SemaphoreType.DMA((2,)),
            pltpu.SemaphoreType.DMA((2,)),
        ],
    )
    def _sc_gather(
        lut_hbm, a0_hbm, a1_hbm, a2_hbm, a3_hbm, out_hbm,
        lut_v, b00, b01, b02, b03, b10, b11, b12, b13,
        rows0, rows1, isem, ssem,
    ):
        ea_bufs = ((b00, b01, b02, b03), (b10, b11, b12, b13))
        col_hbm = (a0_hbm, a1_hbm, a2_hbm, a3_hbm)
        wid = lax.axis_index("s") * NC + lax.axis_index("c")
        base0 = wid * EPW

        pltpu.sync_copy(lut_hbm, lut_v)

        def fetches(j, b):
            return [
                pltpu.make_async_copy(
                    col_hbm[t].at[pl.ds(base0 + j * CHUNK, CHUNK)],
                    ea_bufs[b][t],
                    isem.at[b],
                )
                for t in range(4)
            ]

        def fetch_start(j, b):
            for c in fetches(j, b):
                c.start()

        def fetch_wait(j, b):
            for c in fetches(j, b):
                c.wait()

        def compute(j, b, rows_ref):
            a0, a1, a2, a3 = ea_bufs[b]

            @plsc.parallel_loop(0, CHUNK // 16)
            def body(q):
                s = pl.ds(q * 16, 16)
                cvec = (
                    a0[s] * (24 * EMB)
                    + a1[s] * (4 * EMB)
                    + a2[s] * (2 * EMB)
                    + a3[s] * EMB
                )
                for l in range(16):
                    cb = cvec[l]
                    for k in range(8):
                        rows_ref[q * 16 + l, pl.ds(k * 16, 16)] = lut_v[
                            pl.ds(cb + k * 16, 16)
                        ]

        def scat(j, rows_ref, b):
            return pltpu.make_async_copy(
                rows_ref, out_hbm.at[pl.ds(base0 + j * CHUNK, CHUNK)], ssem.at[b]
            )

        fetch_start(0, 0)

        def step(j, b, rows_ref, last):
            fetch_wait(j, b)
            if not last:
                fetch_start(j + 1, 1 - b)

            @pl.when(j >= 2)
            def _():
                scat(j - 2, rows_ref, b).wait()

            compute(j, b, rows_ref)
            scat(j, rows_ref, b).start()

        def pair(p, carry):
            j0 = 2 * p
            step(j0, 0, rows0, False)
            step(j0 + 1, 1, rows1, False)
            return carry

        lax.fori_loop(0, NCH // 2, pair, 0)

        step(NCH - 1, 0, rows0, True)
        scat(NCH - 2, rows1, 1).wait()
        scat(NCH - 1, rows0, 0).wait()

    return _sc_gather


def kernel(edge_attr, W0, W1, W2, W3):
    lut = _prep(W0, W1, W2, W3)
    ea = edge_attr.astype(jnp.int32)
    cols = [ea[:, t] for t in range(4)]
    return _make_sc_gather()(lut.reshape(NLUT * EMB), *cols)
